# baseline (speedup 1.0000x reference)
import dataclasses
import functools

import jax
import jax.numpy as jnp
from jax import lax
from jax.experimental import pallas as pl
from jax.experimental.pallas import tpu as pltpu
from jax.experimental.pallas import tpu_sc as plsc

_GATHER_DNUMS = lax.GatherDimensionNumbers(
    offset_dims=(), collapsed_slice_dims=(0,), start_index_map=(0,))


def _lane_bcast(p, lj):
    idx = jnp.full((16, 1), lj, jnp.int32)
    return lax.gather(p, idx, _GATHER_DNUMS, slice_sizes=(1,),
                      mode=lax.GatherScatterMode.PROMISE_IN_BOUNDS)


N = 10000
E = 320000
D = 128
N_REL = 200
N_TIME = 366

NC = 2
NS = 16
DH = D // NC
EPW = E // NS
C = 80
BC = 10
NBLK = EPW // (C * BC)
G = C // 16
NPAD = 10240
RPT = NPAD // NS


def _prep_body(x_ref, rel_ref, tim_ref, ah_ref, at_ref, ar_ref, ats_ref,
               h_ref, t_ref, r_ref, ts_ref):
    x = x_ref[...]
    h_ref[...] = jnp.sum(x * ah_ref[...], axis=1)
    t_ref[...] = jnp.sum(x * at_ref[...], axis=1)
    r_ref[...] = jnp.sum(rel_ref[...] * ar_ref[...], axis=1)
    ts_ref[...] = jnp.sum(tim_ref[...] * ats_ref[...], axis=1)


def _prep(x, rel_repr, time_emds, attn_h, attn_t, attn_r, attn_ts):
    return pl.pallas_call(
        _prep_body,
        out_shape=[
            jax.ShapeDtypeStruct((N,), jnp.float32),
            jax.ShapeDtypeStruct((N,), jnp.float32),
            jax.ShapeDtypeStruct((N_REL,), jnp.float32),
            jax.ShapeDtypeStruct((N_TIME,), jnp.float32),
        ],
    )(x, rel_repr, time_emds,
      attn_h.reshape(1, D), attn_t.reshape(1, D),
      attn_r.reshape(1, D), attn_ts.reshape(1, D))


def _sc_body(hatt_hbm, tatt_hbm, ratt_hbm, tsatt_hbm,
             src_hbm, dst_hbm, et_hbm, ets_hbm,
             x2_hbm, rel2_hbm, tim2_hbm,
             acc_out, den_out,
             src_v, dst_v, et_v, ets_v,
             hatt_v, tatt_v, ratt_v, tsatt_v,
             xrA, prA, xrB, prB, rel_t, tim_t, msg_v,
             acc_sp, den_sp, semA, semB, ssemA, ssemB):
    cid = lax.axis_index("c")
    sid = lax.axis_index("s")

    zero16 = jnp.zeros((16,), jnp.float32)
    izero16 = jnp.zeros((16,), jnp.int32)
    xoff_v = jnp.full((16,), cid * N, jnp.int32)

    pltpu.sync_copy(hatt_hbm, hatt_v)
    pltpu.sync_copy(tatt_hbm, tatt_v)
    pltpu.sync_copy(ratt_hbm, ratt_v)
    pltpu.sync_copy(tsatt_hbm, tsatt_v)
    pltpu.sync_copy(rel2_hbm.at[cid], rel_t)
    pltpu.sync_copy(tim2_hbm.at[cid], tim_t)

    @pl.loop(0, C)
    def _(i):
        for v in range(DH // 16):
            xrA[i, pl.ds(v * 16, 16)] = zero16
        prA[i, :] = zero16
        prB[i, :] = zero16

    for k in range(RPT // C):
        off = sid * RPT + k * C
        pltpu.sync_copy(xrA, acc_sp.at[pl.ds(off, C)])
        pltpu.sync_copy(prA, den_sp.at[pl.ds(off, C)])

    plsc.subcore_barrier()

    def issue_p1(jj, xr, sem):
        pltpu.async_copy(x2_hbm.at[src_v.at[jj]], xr, sem)

    def wait_p1(jj, xr, sem):
        pltpu.make_async_copy(x2_hbm.at[src_v.at[jj]], xr, sem).wait()

    def issue_scat(jj, pr, sem):
        pltpu.async_copy(msg_v, acc_sp.at[dst_v.at[jj]], sem, add=True)

        @pl.when(cid == 0)
        def _():
            pltpu.async_copy(pr, den_sp.at[dst_v.at[jj]], sem, add=True)

    def wait_scat(jj, pr, sem):
        pltpu.make_async_copy(msg_v, acc_sp.at[dst_v.at[jj]], sem).wait()

        @pl.when(cid == 0)
        def _():
            pltpu.make_async_copy(pr, den_sp.at[dst_v.at[jj]], sem).wait()

    cols = [jnp.arange(v * 16, v * 16 + 16, dtype=jnp.int32)
            for v in range(DH // 16)]

    def compute_chunk(jj, xr, pr):
        @pl.loop(0, G)
        def _(g):
            b = g * 16
            sl = pl.ds(b, 16)
            eidx = et_v[jj, sl]
            tsidx = ets_v[jj, sl]
            h = plsc.load_gather(hatt_v, [src_v[jj, sl] - xoff_v])
            t = plsc.load_gather(tatt_v, [dst_v[jj, sl]])
            r = plsc.load_gather(ratt_v, [eidx])
            ts = plsc.load_gather(tsatt_v, [tsidx])
            e = h - t + r + ts
            e = jnp.where(e > 0.0, e, 0.1 * e)
            p = jnp.exp(e)
            rows = b + jax.lax.iota(jnp.int32, 16)
            plsc.store_scatter(pr, [rows, izero16], p)
            for lj in range(16):
                row = b + lj
                pb = _lane_bcast(p, lj)
                re = _lane_bcast(eidx, lj)
                te = _lane_bcast(tsidx, lj)
                for v in range(DH // 16):
                    vsl = pl.ds(v * 16, 16)
                    rv = plsc.load_gather(rel_t, [re, cols[v]])
                    tv = plsc.load_gather(tim_t, [te, cols[v]])
                    msg_v[row, vsl] = (
                        (xr[row, vsl] + tv) * (rv + tv) * pb)

    @pl.loop(0, NBLK)
    def _(blk):
        pltpu.sync_copy(src_hbm.at[sid, blk], src_v)
        pltpu.sync_copy(dst_hbm.at[sid, blk], dst_v)
        pltpu.sync_copy(et_hbm.at[sid, blk], et_v)
        pltpu.sync_copy(ets_hbm.at[sid, blk], ets_v)

        @pl.loop(0, BC)
        def _(r):
            for v in range(G):
                sl = pl.ds(v * 16, 16)
                src_v[r, sl] = src_v[r, sl] + xoff_v

        issue_p1(0, xrA, semA)

        @pl.loop(0, BC, step=2)
        def _(jj):
            @pl.when(jj > 0)
            def _():
                wait_scat(jj - 1, prB, ssemB)

            issue_p1(jj + 1, xrB, semB)
            wait_p1(jj, xrA, semA)
            compute_chunk(jj, xrA, prA)
            issue_scat(jj, prA, ssemA)

            wait_scat(jj, prA, ssemA)

            @pl.when(jj + 2 < BC)
            def _():
                issue_p1(jj + 2, xrA, semA)

            wait_p1(jj + 1, xrB, semB)
            compute_chunk(jj + 1, xrB, prB)
            issue_scat(jj + 1, prB, ssemB)

        wait_scat(BC - 1, prB, ssemB)

    plsc.subcore_barrier()

    off = sid * RPT
    pltpu.sync_copy(acc_sp.at[pl.ds(off, RPT)], acc_out.at[cid, pl.ds(off, RPT)])

    @pl.when(cid == 0)
    def _():
        pltpu.sync_copy(den_sp.at[pl.ds(off, RPT)], den_out.at[pl.ds(off, RPT)])


def _sc_edge(hatt, tatt, ratt, tsatt, src4d, dst4d, et4d, ets4d,
             x2, rel2, tim2):
    mesh = plsc.VectorSubcoreMesh(core_axis_name="c", subcore_axis_name="s")
    cp = pltpu.CompilerParams(needs_layout_passes=False,
                              use_tc_tiling_on_sc=False)
    kern = pl.kernel(
        _sc_body,
        out_type=[
            jax.ShapeDtypeStruct((NC, NPAD, DH), jnp.float32),
            jax.ShapeDtypeStruct((NPAD, 16), jnp.float32),
        ],
        mesh=mesh,
        scratch_types=[
            pltpu.VMEM((BC, C), jnp.int32),
            pltpu.VMEM((BC, C), jnp.int32),
            pltpu.VMEM((BC, C), jnp.int32),
            pltpu.VMEM((BC, C), jnp.int32),
            pltpu.VMEM((N,), jnp.float32),
            pltpu.VMEM((N,), jnp.float32),
            pltpu.VMEM((N_REL,), jnp.float32),
            pltpu.VMEM((N_TIME,), jnp.float32),
            pltpu.VMEM((C, DH), jnp.float32),
            pltpu.VMEM((C, 16), jnp.float32),
            pltpu.VMEM((C, DH), jnp.float32),
            pltpu.VMEM((C, 16), jnp.float32),
            pltpu.VMEM((N_REL, DH), jnp.float32),
            pltpu.VMEM((N_TIME, DH), jnp.float32),
            pltpu.VMEM((C, DH), jnp.float32),
            pltpu.VMEM_SHARED((NPAD, DH), jnp.float32),
            pltpu.VMEM_SHARED((NPAD, 16), jnp.float32),
            pltpu.SemaphoreType.DMA,
            pltpu.SemaphoreType.DMA,
            pltpu.SemaphoreType.DMA,
            pltpu.SemaphoreType.DMA,
        ],
        compiler_params=cp,
    )
    return kern(hatt, tatt, ratt, tsatt, src4d, dst4d, et4d, ets4d,
                x2, rel2, tim2)


def _finish_body(acc_ref, den_ref, x_ref, tw_ref, lw_ref, out_ref):
    a = jnp.concatenate([acc_ref[0, :N], acc_ref[1, :N]], axis=1)
    d = den_ref[:N, 0:1]
    d = jnp.where(d <= 0.0, 1.0, d)
    agg = a / d
    out_ref[...] = (
        jnp.dot(agg, tw_ref[...], preferred_element_type=jnp.float32)
        + jnp.dot(x_ref[...], lw_ref[...], preferred_element_type=jnp.float32))


def _finish(acc, den, x, trans_w, loop_w):
    return pl.pallas_call(
        _finish_body,
        out_shape=jax.ShapeDtypeStruct((N, D), jnp.float32),
    )(acc, den, x, trans_w, loop_w)


def _rel_body(rel_ref, w_ref, out_ref):
    out_ref[...] = jnp.dot(rel_ref[...], w_ref[...],
                           preferred_element_type=jnp.float32)


def _rel_out(rel_repr, w_rel):
    return pl.pallas_call(
        _rel_body,
        out_shape=jax.ShapeDtypeStruct((N_REL, D), jnp.float32),
    )(rel_repr, w_rel)


@jax.jit
def kernel(x, edge_index, edge_type, edge_time, rel_repr, time_emds,
           trans_w, loop_w, w_rel, attn_h, attn_t, attn_r, attn_ts):
    hatt, tatt, ratt, tsatt = _prep(
        x, rel_repr, time_emds, attn_h, attn_t, attn_r, attn_ts)

    src4d = edge_index[0].reshape(NS, NBLK, BC, C)
    dst4d = edge_index[1].reshape(NS, NBLK, BC, C)
    et4d = edge_type.reshape(NS, NBLK, BC, C)
    ets4d = edge_time.reshape(NS, NBLK, BC, C)

    x2 = jnp.concatenate([x[:, :DH], x[:, DH:]], axis=0)
    rel2 = jnp.stack([rel_repr[:, :DH], rel_repr[:, DH:]])
    tim2 = jnp.stack([time_emds[:, :DH], time_emds[:, DH:]])

    acc, den = _sc_edge(hatt, tatt, ratt, tsatt,
                        src4d, dst4d, et4d, ets4d,
                        x2, rel2, tim2)

    x_out = _finish(acc, den, x, trans_w, loop_w)
    rel_out = _rel_out(rel_repr, w_rel)
    return (x_out, rel_out)


# speedup vs baseline: 1.8622x; 1.8622x over previous
import dataclasses
import functools

import jax
import jax.numpy as jnp
from jax import lax
from jax.experimental import pallas as pl
from jax.experimental.pallas import tpu as pltpu
from jax.experimental.pallas import tpu_sc as plsc

_GATHER_DNUMS = lax.GatherDimensionNumbers(
    offset_dims=(), collapsed_slice_dims=(0,), start_index_map=(0,))


def _lane_bcast(p, lj):
    idx = jnp.full((16, 1), lj, jnp.int32)
    return lax.gather(p, idx, _GATHER_DNUMS, slice_sizes=(1,),
                      mode=lax.GatherScatterMode.PROMISE_IN_BOUNDS)


N = 10000
E = 320000
D = 128
N_REL = 200
N_TIME = 366

NC = 2
NS = 16
DH = D // NC
EPW = E // NS
C = 80
BC = 10
NBLK = EPW // (C * BC)
G = C // 16
NPAD = 10240
RPT = NPAD // NS


def _prep_body(x_ref, rel_ref, tim_ref, ah_ref, at_ref, ar_ref, ats_ref,
               h_ref, t_ref, r_ref, ts_ref):
    x = x_ref[...]
    h_ref[...] = jnp.sum(x * ah_ref[...], axis=1)
    t_ref[...] = jnp.sum(x * at_ref[...], axis=1)
    r_ref[...] = jnp.sum(rel_ref[...] * ar_ref[...], axis=1)
    ts_ref[...] = jnp.sum(tim_ref[...] * ats_ref[...], axis=1)


def _prep(x, rel_repr, time_emds, attn_h, attn_t, attn_r, attn_ts):
    return pl.pallas_call(
        _prep_body,
        out_shape=[
            jax.ShapeDtypeStruct((N,), jnp.float32),
            jax.ShapeDtypeStruct((N,), jnp.float32),
            jax.ShapeDtypeStruct((N_REL,), jnp.float32),
            jax.ShapeDtypeStruct((N_TIME,), jnp.float32),
        ],
    )(x, rel_repr, time_emds,
      attn_h.reshape(1, D), attn_t.reshape(1, D),
      attn_r.reshape(1, D), attn_ts.reshape(1, D))


def _sc_body(hatt_hbm, tatt_hbm, ratt_hbm, tsatt_hbm,
             src_hbm, dst_hbm, et_hbm, ets_hbm,
             x2_hbm, rel2_hbm, tim2_hbm,
             acc_out, den_out,
             src_v, dst_v, et_v, ets_v,
             hatt_v, tatt_v, ratt_v, tsatt_v,
             xrA, rrA, ttA, prA, msgA, xrB, rrB, ttB, prB, msgB,
             acc_sp, den_sp, semA, semB, ssemA, ssemB):
    cid = lax.axis_index("c")
    sid = lax.axis_index("s")

    zero16 = jnp.zeros((16,), jnp.float32)
    izero16 = jnp.zeros((16,), jnp.int32)
    xoff_v = jnp.full((16,), cid * N, jnp.int32)
    roff_v = jnp.full((16,), cid * N_REL, jnp.int32)
    toff_v = jnp.full((16,), cid * N_TIME, jnp.int32)

    pltpu.sync_copy(hatt_hbm, hatt_v)
    pltpu.sync_copy(tatt_hbm, tatt_v)
    pltpu.sync_copy(ratt_hbm, ratt_v)
    pltpu.sync_copy(tsatt_hbm, tsatt_v)

    @pl.loop(0, C)
    def _(i):
        for v in range(DH // 16):
            xrA[i, pl.ds(v * 16, 16)] = zero16
        prA[i, :] = zero16
        prB[i, :] = zero16

    for k in range(RPT // C):
        off = sid * RPT + k * C
        pltpu.sync_copy(xrA, acc_sp.at[pl.ds(off, C)])
        pltpu.sync_copy(prA, den_sp.at[pl.ds(off, C)])

    plsc.subcore_barrier()

    def issue_p1(jj, xr, rr, tt, sem):
        pltpu.async_copy(x2_hbm.at[src_v.at[jj]], xr, sem)
        pltpu.async_copy(rel2_hbm.at[et_v.at[jj]], rr, sem)
        pltpu.async_copy(tim2_hbm.at[ets_v.at[jj]], tt, sem)

    def wait_p1(jj, xr, rr, tt, sem):
        pltpu.make_async_copy(x2_hbm.at[src_v.at[jj]], xr, sem).wait()
        pltpu.make_async_copy(rel2_hbm.at[et_v.at[jj]], rr, sem).wait()
        pltpu.make_async_copy(tim2_hbm.at[ets_v.at[jj]], tt, sem).wait()

    def issue_scat(jj, msg, pr, sem):
        pltpu.async_copy(msg, acc_sp.at[dst_v.at[jj]], sem, add=True)

        @pl.when(cid == 0)
        def _():
            pltpu.async_copy(pr, den_sp.at[dst_v.at[jj]], sem, add=True)

    def wait_scat(jj, msg, pr, sem):
        pltpu.make_async_copy(msg, acc_sp.at[dst_v.at[jj]], sem).wait()

        @pl.when(cid == 0)
        def _():
            pltpu.make_async_copy(pr, den_sp.at[dst_v.at[jj]], sem).wait()

    def compute_chunk(jj, xr, rr, tt, msg, pr):
        @pl.loop(0, G)
        def _(g):
            b = g * 16
            sl = pl.ds(b, 16)
            h = plsc.load_gather(hatt_v, [src_v[jj, sl] - xoff_v])
            t = plsc.load_gather(tatt_v, [dst_v[jj, sl]])
            r = plsc.load_gather(ratt_v, [et_v[jj, sl] - roff_v])
            ts = plsc.load_gather(tsatt_v, [ets_v[jj, sl] - toff_v])
            e = h - t + r + ts
            e = jnp.where(e > 0.0, e, 0.1 * e)
            p = jnp.exp(e)
            rows = b + jax.lax.iota(jnp.int32, 16)
            plsc.store_scatter(pr, [rows, izero16], p)
            for lj in range(16):
                row = b + lj
                pb = _lane_bcast(p, lj)
                for v in range(DH // 16):
                    vsl = pl.ds(v * 16, 16)
                    tv = tt[row, vsl]
                    msg[row, vsl] = (
                        (xr[row, vsl] + tv) * (rr[row, vsl] + tv) * pb)

    @pl.loop(0, NBLK)
    def _(blk):
        pltpu.sync_copy(src_hbm.at[sid, blk], src_v)
        pltpu.sync_copy(dst_hbm.at[sid, blk], dst_v)
        pltpu.sync_copy(et_hbm.at[sid, blk], et_v)
        pltpu.sync_copy(ets_hbm.at[sid, blk], ets_v)

        @pl.loop(0, BC)
        def _(r):
            for v in range(G):
                sl = pl.ds(v * 16, 16)
                src_v[r, sl] = src_v[r, sl] + xoff_v
                et_v[r, sl] = et_v[r, sl] + roff_v
                ets_v[r, sl] = ets_v[r, sl] + toff_v

        issue_p1(0, xrA, rrA, ttA, semA)

        @pl.loop(0, BC, step=2)
        def _(jj):
            @pl.when(jj > 0)
            def _():
                wait_scat(jj - 2, msgA, prA, ssemA)

            issue_p1(jj + 1, xrB, rrB, ttB, semB)
            wait_p1(jj, xrA, rrA, ttA, semA)
            compute_chunk(jj, xrA, rrA, ttA, msgA, prA)
            issue_scat(jj, msgA, prA, ssemA)

            @pl.when(jj > 0)
            def _():
                wait_scat(jj - 1, msgB, prB, ssemB)

            @pl.when(jj + 2 < BC)
            def _():
                issue_p1(jj + 2, xrA, rrA, ttA, semA)

            wait_p1(jj + 1, xrB, rrB, ttB, semB)
            compute_chunk(jj + 1, xrB, rrB, ttB, msgB, prB)
            issue_scat(jj + 1, msgB, prB, ssemB)

        wait_scat(BC - 2, msgA, prA, ssemA)
        wait_scat(BC - 1, msgB, prB, ssemB)

    plsc.subcore_barrier()

    off = sid * RPT
    pltpu.sync_copy(acc_sp.at[pl.ds(off, RPT)], acc_out.at[cid, pl.ds(off, RPT)])

    @pl.when(cid == 0)
    def _():
        pltpu.sync_copy(den_sp.at[pl.ds(off, RPT)], den_out.at[pl.ds(off, RPT)])


def _sc_edge(hatt, tatt, ratt, tsatt, src4d, dst4d, et4d, ets4d,
             x2, rel2, tim2):
    mesh = plsc.VectorSubcoreMesh(core_axis_name="c", subcore_axis_name="s")
    cp = pltpu.CompilerParams(needs_layout_passes=False,
                              use_tc_tiling_on_sc=False)
    kern = pl.kernel(
        _sc_body,
        out_type=[
            jax.ShapeDtypeStruct((NC, NPAD, DH), jnp.float32),
            jax.ShapeDtypeStruct((NPAD, 16), jnp.float32),
        ],
        mesh=mesh,
        scratch_types=[
            pltpu.VMEM((BC, C), jnp.int32),
            pltpu.VMEM((BC, C), jnp.int32),
            pltpu.VMEM((BC, C), jnp.int32),
            pltpu.VMEM((BC, C), jnp.int32),
            pltpu.VMEM((N,), jnp.float32),
            pltpu.VMEM((N,), jnp.float32),
            pltpu.VMEM((N_REL,), jnp.float32),
            pltpu.VMEM((N_TIME,), jnp.float32),
            pltpu.VMEM((C, DH), jnp.float32),
            pltpu.VMEM((C, DH), jnp.float32),
            pltpu.VMEM((C, DH), jnp.float32),
            pltpu.VMEM((C, 16), jnp.float32),
            pltpu.VMEM((C, DH), jnp.float32),
            pltpu.VMEM((C, DH), jnp.float32),
            pltpu.VMEM((C, DH), jnp.float32),
            pltpu.VMEM((C, DH), jnp.float32),
            pltpu.VMEM((C, 16), jnp.float32),
            pltpu.VMEM((C, DH), jnp.float32),
            pltpu.VMEM_SHARED((NPAD, DH), jnp.float32),
            pltpu.VMEM_SHARED((NPAD, 16), jnp.float32),
            pltpu.SemaphoreType.DMA,
            pltpu.SemaphoreType.DMA,
            pltpu.SemaphoreType.DMA,
            pltpu.SemaphoreType.DMA,
        ],
        compiler_params=cp,
    )
    return kern(hatt, tatt, ratt, tsatt, src4d, dst4d, et4d, ets4d,
                x2, rel2, tim2)


def _finish_body(acc_ref, den_ref, x_ref, tw_ref, lw_ref, out_ref):
    a = jnp.concatenate([acc_ref[0, :N], acc_ref[1, :N]], axis=1)
    d = den_ref[:N, 0:1]
    d = jnp.where(d <= 0.0, 1.0, d)
    agg = a / d
    out_ref[...] = (
        jnp.dot(agg, tw_ref[...], preferred_element_type=jnp.float32)
        + jnp.dot(x_ref[...], lw_ref[...], preferred_element_type=jnp.float32))


def _finish(acc, den, x, trans_w, loop_w):
    return pl.pallas_call(
        _finish_body,
        out_shape=jax.ShapeDtypeStruct((N, D), jnp.float32),
    )(acc, den, x, trans_w, loop_w)


def _rel_body(rel_ref, w_ref, out_ref):
    out_ref[...] = jnp.dot(rel_ref[...], w_ref[...],
                           preferred_element_type=jnp.float32)


def _rel_out(rel_repr, w_rel):
    return pl.pallas_call(
        _rel_body,
        out_shape=jax.ShapeDtypeStruct((N_REL, D), jnp.float32),
    )(rel_repr, w_rel)


@jax.jit
def kernel(x, edge_index, edge_type, edge_time, rel_repr, time_emds,
           trans_w, loop_w, w_rel, attn_h, attn_t, attn_r, attn_ts):
    hatt, tatt, ratt, tsatt = _prep(
        x, rel_repr, time_emds, attn_h, attn_t, attn_r, attn_ts)

    src4d = edge_index[0].reshape(NS, NBLK, BC, C)
    dst4d = edge_index[1].reshape(NS, NBLK, BC, C)
    et4d = edge_type.reshape(NS, NBLK, BC, C)
    ets4d = edge_time.reshape(NS, NBLK, BC, C)

    x2 = jnp.concatenate([x[:, :DH], x[:, DH:]], axis=0)
    rel2 = jnp.concatenate([rel_repr[:, :DH], rel_repr[:, DH:]], axis=0)
    tim2 = jnp.concatenate([time_emds[:, :DH], time_emds[:, DH:]], axis=0)

    acc, den = _sc_edge(hatt, tatt, ratt, tsatt,
                        src4d, dst4d, et4d, ets4d,
                        x2, rel2, tim2)

    x_out = _finish(acc, den, x, trans_w, loop_w)
    rel_out = _rel_out(rel_repr, w_rel)
    return (x_out, rel_out)


# speedup vs baseline: 2.0163x; 1.0827x over previous
import dataclasses
import functools

import jax
import jax.numpy as jnp
from jax import lax
from jax.experimental import pallas as pl
from jax.experimental.pallas import tpu as pltpu
from jax.experimental.pallas import tpu_sc as plsc

_GATHER_DNUMS = lax.GatherDimensionNumbers(
    offset_dims=(), collapsed_slice_dims=(0,), start_index_map=(0,))


def _lane_bcast(p, lj):
    idx = jnp.full((16, 1), lj, jnp.int32)
    return lax.gather(p, idx, _GATHER_DNUMS, slice_sizes=(1,),
                      mode=lax.GatherScatterMode.PROMISE_IN_BOUNDS)


N = 10000
E = 320000
D = 128
N_REL = 200
N_TIME = 366

NC = 2
NS = 16
DH = D // NC
EPW = E // NS
C = 80
BC = 10
NBLK = EPW // (C * BC)
G = C // 16
NPAD = 10240
RPT = NPAD // NS


def _prep_body(x_ref, rel_ref, tim_ref, ah_ref, at_ref, ar_ref, ats_ref,
               h_ref, t_ref, r_ref, ts_ref):
    x = x_ref[...]
    h_ref[...] = jnp.sum(x * ah_ref[...], axis=1)
    t_ref[...] = jnp.sum(x * at_ref[...], axis=1)
    r_ref[...] = jnp.sum(rel_ref[...] * ar_ref[...], axis=1)
    ts_ref[...] = jnp.sum(tim_ref[...] * ats_ref[...], axis=1)


def _prep(x, rel_repr, time_emds, attn_h, attn_t, attn_r, attn_ts):
    return pl.pallas_call(
        _prep_body,
        out_shape=[
            jax.ShapeDtypeStruct((N,), jnp.float32),
            jax.ShapeDtypeStruct((N,), jnp.float32),
            jax.ShapeDtypeStruct((N_REL,), jnp.float32),
            jax.ShapeDtypeStruct((N_TIME,), jnp.float32),
        ],
    )(x, rel_repr, time_emds,
      attn_h.reshape(1, D), attn_t.reshape(1, D),
      attn_r.reshape(1, D), attn_ts.reshape(1, D))


def _sc_body(hatt_hbm, tatt_hbm, ratt_hbm, tsatt_hbm,
             src_hbm, dst_hbm, et_hbm, ets_hbm,
             x2_hbm, rel2_hbm, tim2_hbm,
             acc_out, den_out,
             src_v, dst_v, et_v, ets_v,
             hatt_v, tatt_v, ratt_v, tsatt_v,
             xrA, rrA, ttA, prA, msgA, xrB, rrB, ttB, prB, msgB,
             acc_sp, den_sp, semA, semB, ssemA, ssemB):
    cid = lax.axis_index("c")
    sid = lax.axis_index("s")

    zero16 = jnp.zeros((16,), jnp.float32)
    izero16 = jnp.zeros((16,), jnp.int32)
    xoff_v = jnp.full((16,), cid * N, jnp.int32)
    roff_v = jnp.full((16,), cid * N_REL, jnp.int32)
    toff_v = jnp.full((16,), cid * N_TIME, jnp.int32)

    pltpu.sync_copy(hatt_hbm, hatt_v)
    pltpu.sync_copy(tatt_hbm, tatt_v)
    pltpu.sync_copy(ratt_hbm, ratt_v)
    pltpu.sync_copy(tsatt_hbm, tsatt_v)

    @pl.loop(0, C)
    def _(i):
        for v in range(DH // 16):
            xrA[i, pl.ds(v * 16, 16)] = zero16
        prA[i, :] = zero16
        prB[i, :] = zero16

    for k in range(RPT // C):
        off = sid * RPT + k * C
        pltpu.sync_copy(xrA, acc_sp.at[pl.ds(off, C)])
        pltpu.sync_copy(prA, den_sp.at[pl.ds(off, C)])

    plsc.subcore_barrier()

    def issue_p1(jj, xr, rr, tt, sem):
        pltpu.async_copy(x2_hbm.at[src_v.at[jj]], xr, sem)
        pltpu.async_copy(rel2_hbm.at[et_v.at[jj]], rr, sem)
        pltpu.async_copy(tim2_hbm.at[ets_v.at[jj]], tt, sem)

    def wait_p1(jj, xr, rr, tt, sem):
        pltpu.make_async_copy(x2_hbm.at[src_v.at[jj]], xr, sem).wait()
        pltpu.make_async_copy(rel2_hbm.at[et_v.at[jj]], rr, sem).wait()
        pltpu.make_async_copy(tim2_hbm.at[ets_v.at[jj]], tt, sem).wait()

    def issue_scat(jj, msg, pr, sem):
        pltpu.async_copy(msg, acc_sp.at[dst_v.at[jj]], sem, add=True)

        @pl.when(cid == 0)
        def _():
            pltpu.async_copy(pr, den_sp.at[dst_v.at[jj]], sem, add=True)

    def wait_scat(jj, msg, pr, sem):
        pltpu.make_async_copy(msg, acc_sp.at[dst_v.at[jj]], sem).wait()

        @pl.when(cid == 0)
        def _():
            pltpu.make_async_copy(pr, den_sp.at[dst_v.at[jj]], sem).wait()

    def compute_chunk(jj, xr, rr, tt, msg, pr):
        @pl.loop(0, G)
        def _(g):
            b = g * 16
            sl = pl.ds(b, 16)
            h = plsc.load_gather(hatt_v, [src_v[jj, sl] - xoff_v])
            t = plsc.load_gather(tatt_v, [dst_v[jj, sl]])
            r = plsc.load_gather(ratt_v, [et_v[jj, sl] - roff_v])
            ts = plsc.load_gather(tsatt_v, [ets_v[jj, sl] - toff_v])
            e = h - t + r + ts
            e = jnp.where(e > 0.0, e, 0.1 * e)
            p = jnp.exp(e)
            rows = b + jax.lax.iota(jnp.int32, 16)
            plsc.store_scatter(pr, [rows, izero16], p)
            for lj in range(16):
                row = b + lj
                pb = _lane_bcast(p, lj)
                for v in range(DH // 16):
                    vsl = pl.ds(v * 16, 16)
                    tv = tt[row, vsl]
                    msg[row, vsl] = (
                        (xr[row, vsl] + tv) * (rr[row, vsl] + tv) * pb)

    @pl.loop(0, NBLK)
    def _(blk):
        i1 = pltpu.async_copy(src_hbm.at[sid, blk], src_v, semA)
        i2 = pltpu.async_copy(dst_hbm.at[sid, blk], dst_v, semA)
        i3 = pltpu.async_copy(et_hbm.at[sid, blk], et_v, semA)
        i4 = pltpu.async_copy(ets_hbm.at[sid, blk], ets_v, semA)
        i1.wait()
        i2.wait()
        i3.wait()
        i4.wait()

        @pl.loop(0, BC)
        def _(r):
            for v in range(G):
                sl = pl.ds(v * 16, 16)
                src_v[r, sl] = src_v[r, sl] + xoff_v
                et_v[r, sl] = et_v[r, sl] + roff_v
                ets_v[r, sl] = ets_v[r, sl] + toff_v

        issue_p1(0, xrA, rrA, ttA, semA)

        @pl.loop(0, BC, step=2)
        def _(jj):
            @pl.when(jj > 0)
            def _():
                wait_scat(jj - 2, msgA, prA, ssemA)

            issue_p1(jj + 1, xrB, rrB, ttB, semB)
            wait_p1(jj, xrA, rrA, ttA, semA)
            compute_chunk(jj, xrA, rrA, ttA, msgA, prA)
            issue_scat(jj, msgA, prA, ssemA)

            @pl.when(jj > 0)
            def _():
                wait_scat(jj - 1, msgB, prB, ssemB)

            @pl.when(jj + 2 < BC)
            def _():
                issue_p1(jj + 2, xrA, rrA, ttA, semA)

            wait_p1(jj + 1, xrB, rrB, ttB, semB)
            compute_chunk(jj + 1, xrB, rrB, ttB, msgB, prB)
            issue_scat(jj + 1, msgB, prB, ssemB)

        wait_scat(BC - 2, msgA, prA, ssemA)
        wait_scat(BC - 1, msgB, prB, ssemB)

    plsc.subcore_barrier()

    off = sid * RPT
    pltpu.sync_copy(acc_sp.at[pl.ds(off, RPT)], acc_out.at[cid, pl.ds(off, RPT)])

    @pl.when(cid == 0)
    def _():
        pltpu.sync_copy(den_sp.at[pl.ds(off, RPT)], den_out.at[pl.ds(off, RPT)])


def _sc_edge(hatt, tatt, ratt, tsatt, src4d, dst4d, et4d, ets4d,
             x2, rel2, tim2):
    mesh = plsc.VectorSubcoreMesh(core_axis_name="c", subcore_axis_name="s")
    cp = pltpu.CompilerParams(needs_layout_passes=False,
                              use_tc_tiling_on_sc=False)
    kern = pl.kernel(
        _sc_body,
        out_type=[
            jax.ShapeDtypeStruct((NC, NPAD, DH), jnp.float32),
            jax.ShapeDtypeStruct((NPAD, 16), jnp.float32),
        ],
        mesh=mesh,
        scratch_types=[
            pltpu.VMEM((BC, C), jnp.int32),
            pltpu.VMEM((BC, C), jnp.int32),
            pltpu.VMEM((BC, C), jnp.int32),
            pltpu.VMEM((BC, C), jnp.int32),
            pltpu.VMEM((N,), jnp.float32),
            pltpu.VMEM((N,), jnp.float32),
            pltpu.VMEM((N_REL,), jnp.float32),
            pltpu.VMEM((N_TIME,), jnp.float32),
            pltpu.VMEM((C, DH), jnp.float32),
            pltpu.VMEM((C, DH), jnp.float32),
            pltpu.VMEM((C, DH), jnp.float32),
            pltpu.VMEM((C, 16), jnp.float32),
            pltpu.VMEM((C, DH), jnp.float32),
            pltpu.VMEM((C, DH), jnp.float32),
            pltpu.VMEM((C, DH), jnp.float32),
            pltpu.VMEM((C, DH), jnp.float32),
            pltpu.VMEM((C, 16), jnp.float32),
            pltpu.VMEM((C, DH), jnp.float32),
            pltpu.VMEM_SHARED((NPAD, DH), jnp.float32),
            pltpu.VMEM_SHARED((NPAD, 16), jnp.float32),
            pltpu.SemaphoreType.DMA,
            pltpu.SemaphoreType.DMA,
            pltpu.SemaphoreType.DMA,
            pltpu.SemaphoreType.DMA,
        ],
        compiler_params=cp,
    )
    return kern(hatt, tatt, ratt, tsatt, src4d, dst4d, et4d, ets4d,
                x2, rel2, tim2)


def _finish_body(acc_ref, den_ref, x_ref, tw_ref, lw_ref, out_ref):
    a = jnp.concatenate([acc_ref[0, :N], acc_ref[1, :N]], axis=1)
    d = den_ref[:N, 0:1]
    d = jnp.where(d <= 0.0, 1.0, d)
    agg = a / d
    out_ref[...] = (
        jnp.dot(agg, tw_ref[...], preferred_element_type=jnp.float32)
        + jnp.dot(x_ref[...], lw_ref[...], preferred_element_type=jnp.float32))


def _finish(acc, den, x, trans_w, loop_w):
    return pl.pallas_call(
        _finish_body,
        out_shape=jax.ShapeDtypeStruct((N, D), jnp.float32),
    )(acc, den, x, trans_w, loop_w)


def _rel_body(rel_ref, w_ref, out_ref):
    out_ref[...] = jnp.dot(rel_ref[...], w_ref[...],
                           preferred_element_type=jnp.float32)


def _rel_out(rel_repr, w_rel):
    return pl.pallas_call(
        _rel_body,
        out_shape=jax.ShapeDtypeStruct((N_REL, D), jnp.float32),
    )(rel_repr, w_rel)


@jax.jit
def kernel(x, edge_index, edge_type, edge_time, rel_repr, time_emds,
           trans_w, loop_w, w_rel, attn_h, attn_t, attn_r, attn_ts):
    hatt, tatt, ratt, tsatt = _prep(
        x, rel_repr, time_emds, attn_h, attn_t, attn_r, attn_ts)

    src4d = edge_index[0].reshape(NS, NBLK, BC, C)
    dst4d = edge_index[1].reshape(NS, NBLK, BC, C)
    et4d = edge_type.reshape(NS, NBLK, BC, C)
    ets4d = edge_time.reshape(NS, NBLK, BC, C)

    x2 = jnp.concatenate([x[:, :DH], x[:, DH:]], axis=0)
    rel2 = jnp.concatenate([rel_repr[:, :DH], rel_repr[:, DH:]], axis=0)
    tim2 = jnp.concatenate([time_emds[:, :DH], time_emds[:, DH:]], axis=0)

    acc, den = _sc_edge(hatt, tatt, ratt, tsatt,
                        src4d, dst4d, et4d, ets4d,
                        x2, rel2, tim2)

    x_out = _finish(acc, den, x, trans_w, loop_w)
    rel_out = _rel_out(rel_repr, w_rel)
    return (x_out, rel_out)


# speedup vs baseline: 2.0311x; 1.0073x over previous
import dataclasses
import functools

import jax
import jax.numpy as jnp
from jax import lax
from jax.experimental import pallas as pl
from jax.experimental.pallas import tpu as pltpu
from jax.experimental.pallas import tpu_sc as plsc

_GATHER_DNUMS = lax.GatherDimensionNumbers(
    offset_dims=(), collapsed_slice_dims=(0,), start_index_map=(0,))


def _lane_bcast(p, lj):
    idx = jnp.full((16, 1), lj, jnp.int32)
    return lax.gather(p, idx, _GATHER_DNUMS, slice_sizes=(1,),
                      mode=lax.GatherScatterMode.PROMISE_IN_BOUNDS)


N = 10000
E = 320000
D = 128
N_REL = 200
N_TIME = 366

NC = 2
NS = 16
DH = D // NC
EPW = E // NS
C = 80
BC = 10
NBLK = EPW // (C * BC)
G = C // 16
NPAD = 10240
RPT = NPAD // NS


def _prep_body(x_ref, rel_ref, tim_ref, ah_ref, at_ref, ar_ref, ats_ref,
               wr_ref, h_ref, t_ref, r_ref, ts_ref, ro_ref):
    x = x_ref[...]
    h_ref[...] = jnp.sum(x * ah_ref[...], axis=1)
    t_ref[...] = jnp.sum(x * at_ref[...], axis=1)
    r_ref[...] = jnp.sum(rel_ref[...] * ar_ref[...], axis=1)
    ts_ref[...] = jnp.sum(tim_ref[...] * ats_ref[...], axis=1)
    ro_ref[...] = jnp.dot(rel_ref[...], wr_ref[...],
                          preferred_element_type=jnp.float32)


def _prep(x, rel_repr, time_emds, attn_h, attn_t, attn_r, attn_ts, w_rel):
    return pl.pallas_call(
        _prep_body,
        out_shape=[
            jax.ShapeDtypeStruct((N,), jnp.float32),
            jax.ShapeDtypeStruct((N,), jnp.float32),
            jax.ShapeDtypeStruct((N_REL,), jnp.float32),
            jax.ShapeDtypeStruct((N_TIME,), jnp.float32),
            jax.ShapeDtypeStruct((N_REL, D), jnp.float32),
        ],
    )(x, rel_repr, time_emds,
      attn_h.reshape(1, D), attn_t.reshape(1, D),
      attn_r.reshape(1, D), attn_ts.reshape(1, D), w_rel)


def _sc_body(hatt_hbm, tatt_hbm, ratt_hbm, tsatt_hbm,
             src_hbm, dst_hbm, et_hbm, ets_hbm,
             x2_hbm, rel2_hbm, tim2_hbm,
             acc_out, den_out,
             src_v, dst_v, et_v, ets_v,
             hatt_v, tatt_v, ratt_v, tsatt_v,
             xrA, rrA, ttA, prA, msgA, xrB, rrB, ttB, prB, msgB,
             acc_sp, den_sp, semA, semB, ssemA, ssemB):
    cid = lax.axis_index("c")
    sid = lax.axis_index("s")

    zero16 = jnp.zeros((16,), jnp.float32)
    izero16 = jnp.zeros((16,), jnp.int32)
    xoff_v = jnp.full((16,), cid * N, jnp.int32)
    roff_v = jnp.full((16,), cid * N_REL, jnp.int32)
    toff_v = jnp.full((16,), cid * N_TIME, jnp.int32)

    t1 = pltpu.async_copy(hatt_hbm, hatt_v, semB)
    t2 = pltpu.async_copy(tatt_hbm, tatt_v, semB)
    t3 = pltpu.async_copy(ratt_hbm, ratt_v, semB)
    t4 = pltpu.async_copy(tsatt_hbm, tsatt_v, semB)

    @pl.loop(0, C)
    def _(i):
        for v in range(DH // 16):
            xrA[i, pl.ds(v * 16, 16)] = zero16
        prA[i, :] = zero16
        prB[i, :] = zero16

    zcopies = []
    for k in range(RPT // C):
        off = sid * RPT + k * C
        zcopies.append(pltpu.async_copy(xrA, acc_sp.at[pl.ds(off, C)], ssemA))
        zcopies.append(pltpu.async_copy(prA, den_sp.at[pl.ds(off, C)], ssemB))
    for zc in zcopies:
        zc.wait()
    t1.wait()
    t2.wait()
    t3.wait()
    t4.wait()

    plsc.subcore_barrier()

    def issue_p1(jj, xr, rr, tt, sem):
        pltpu.async_copy(x2_hbm.at[src_v.at[jj]], xr, sem)
        pltpu.async_copy(rel2_hbm.at[et_v.at[jj]], rr, sem)
        pltpu.async_copy(tim2_hbm.at[ets_v.at[jj]], tt, sem)

    def wait_p1(jj, xr, rr, tt, sem):
        pltpu.make_async_copy(x2_hbm.at[src_v.at[jj]], xr, sem).wait()
        pltpu.make_async_copy(rel2_hbm.at[et_v.at[jj]], rr, sem).wait()
        pltpu.make_async_copy(tim2_hbm.at[ets_v.at[jj]], tt, sem).wait()

    def issue_scat(jj, msg, pr, sem):
        pltpu.async_copy(msg, acc_sp.at[dst_v.at[jj]], sem, add=True)

        @pl.when(cid == 0)
        def _():
            pltpu.async_copy(pr, den_sp.at[dst_v.at[jj]], sem, add=True)

    def wait_scat(jj, msg, pr, sem):
        pltpu.make_async_copy(msg, acc_sp.at[dst_v.at[jj]], sem).wait()

        @pl.when(cid == 0)
        def _():
            pltpu.make_async_copy(pr, den_sp.at[dst_v.at[jj]], sem).wait()

    def compute_chunk(jj, xr, rr, tt, msg, pr):
        @pl.loop(0, G)
        def _(g):
            b = g * 16
            sl = pl.ds(b, 16)
            h = plsc.load_gather(hatt_v, [src_v[jj, sl] - xoff_v])
            t = plsc.load_gather(tatt_v, [dst_v[jj, sl]])
            r = plsc.load_gather(ratt_v, [et_v[jj, sl] - roff_v])
            ts = plsc.load_gather(tsatt_v, [ets_v[jj, sl] - toff_v])
            e = h - t + r + ts
            e = jnp.where(e > 0.0, e, 0.1 * e)
            p = jnp.exp(e)
            rows = b + jax.lax.iota(jnp.int32, 16)
            plsc.store_scatter(pr, [rows, izero16], p)
            for lj in range(16):
                row = b + lj
                pb = _lane_bcast(p, lj)
                for v in range(DH // 16):
                    vsl = pl.ds(v * 16, 16)
                    tv = tt[row, vsl]
                    msg[row, vsl] = (
                        (xr[row, vsl] + tv) * (rr[row, vsl] + tv) * pb)

    @pl.loop(0, NBLK)
    def _(blk):
        i1 = pltpu.async_copy(src_hbm.at[sid, blk], src_v, semA)
        i2 = pltpu.async_copy(dst_hbm.at[sid, blk], dst_v, semA)
        i3 = pltpu.async_copy(et_hbm.at[sid, blk], et_v, semA)
        i4 = pltpu.async_copy(ets_hbm.at[sid, blk], ets_v, semA)
        i1.wait()
        i2.wait()
        i3.wait()
        i4.wait()

        @pl.loop(0, BC)
        def _(r):
            for v in range(G):
                sl = pl.ds(v * 16, 16)
                src_v[r, sl] = src_v[r, sl] + xoff_v
                et_v[r, sl] = et_v[r, sl] + roff_v
                ets_v[r, sl] = ets_v[r, sl] + toff_v

        issue_p1(0, xrA, rrA, ttA, semA)

        @pl.loop(0, BC, step=2)
        def _(jj):
            @pl.when(jj > 0)
            def _():
                wait_scat(jj - 2, msgA, prA, ssemA)

            issue_p1(jj + 1, xrB, rrB, ttB, semB)
            wait_p1(jj, xrA, rrA, ttA, semA)
            compute_chunk(jj, xrA, rrA, ttA, msgA, prA)
            issue_scat(jj, msgA, prA, ssemA)

            @pl.when(jj > 0)
            def _():
                wait_scat(jj - 1, msgB, prB, ssemB)

            @pl.when(jj + 2 < BC)
            def _():
                issue_p1(jj + 2, xrA, rrA, ttA, semA)

            wait_p1(jj + 1, xrB, rrB, ttB, semB)
            compute_chunk(jj + 1, xrB, rrB, ttB, msgB, prB)
            issue_scat(jj + 1, msgB, prB, ssemB)

        wait_scat(BC - 2, msgA, prA, ssemA)
        wait_scat(BC - 1, msgB, prB, ssemB)

    plsc.subcore_barrier()

    off = sid * RPT
    pltpu.sync_copy(acc_sp.at[pl.ds(off, RPT)], acc_out.at[cid, pl.ds(off, RPT)])

    @pl.when(cid == 0)
    def _():
        pltpu.sync_copy(den_sp.at[pl.ds(off, RPT)], den_out.at[pl.ds(off, RPT)])


def _sc_edge(hatt, tatt, ratt, tsatt, src4d, dst4d, et4d, ets4d,
             x2, rel2, tim2):
    mesh = plsc.VectorSubcoreMesh(core_axis_name="c", subcore_axis_name="s")
    cp = pltpu.CompilerParams(needs_layout_passes=False,
                              use_tc_tiling_on_sc=False)
    kern = pl.kernel(
        _sc_body,
        out_type=[
            jax.ShapeDtypeStruct((NC, NPAD, DH), jnp.float32),
            jax.ShapeDtypeStruct((NPAD, 16), jnp.float32),
        ],
        mesh=mesh,
        scratch_types=[
            pltpu.VMEM((BC, C), jnp.int32),
            pltpu.VMEM((BC, C), jnp.int32),
            pltpu.VMEM((BC, C), jnp.int32),
            pltpu.VMEM((BC, C), jnp.int32),
            pltpu.VMEM((N,), jnp.float32),
            pltpu.VMEM((N,), jnp.float32),
            pltpu.VMEM((N_REL,), jnp.float32),
            pltpu.VMEM((N_TIME,), jnp.float32),
            pltpu.VMEM((C, DH), jnp.float32),
            pltpu.VMEM((C, DH), jnp.float32),
            pltpu.VMEM((C, DH), jnp.float32),
            pltpu.VMEM((C, 16), jnp.float32),
            pltpu.VMEM((C, DH), jnp.float32),
            pltpu.VMEM((C, DH), jnp.float32),
            pltpu.VMEM((C, DH), jnp.float32),
            pltpu.VMEM((C, DH), jnp.float32),
            pltpu.VMEM((C, 16), jnp.float32),
            pltpu.VMEM((C, DH), jnp.float32),
            pltpu.VMEM_SHARED((NPAD, DH), jnp.float32),
            pltpu.VMEM_SHARED((NPAD, 16), jnp.float32),
            pltpu.SemaphoreType.DMA,
            pltpu.SemaphoreType.DMA,
            pltpu.SemaphoreType.DMA,
            pltpu.SemaphoreType.DMA,
        ],
        compiler_params=cp,
    )
    return kern(hatt, tatt, ratt, tsatt, src4d, dst4d, et4d, ets4d,
                x2, rel2, tim2)


def _finish_body(acc_ref, den_ref, x_ref, tw_ref, lw_ref, out_ref):
    a = jnp.concatenate([acc_ref[0, :N], acc_ref[1, :N]], axis=1)
    d = den_ref[:N, 0:1]
    d = jnp.where(d <= 0.0, 1.0, d)
    agg = a / d
    out_ref[...] = (
        jnp.dot(agg, tw_ref[...], preferred_element_type=jnp.float32)
        + jnp.dot(x_ref[...], lw_ref[...], preferred_element_type=jnp.float32))


def _finish(acc, den, x, trans_w, loop_w):
    return pl.pallas_call(
        _finish_body,
        out_shape=jax.ShapeDtypeStruct((N, D), jnp.float32),
    )(acc, den, x, trans_w, loop_w)


@jax.jit
def kernel(x, edge_index, edge_type, edge_time, rel_repr, time_emds,
           trans_w, loop_w, w_rel, attn_h, attn_t, attn_r, attn_ts):
    hatt, tatt, ratt, tsatt, rel_out = _prep(
        x, rel_repr, time_emds, attn_h, attn_t, attn_r, attn_ts, w_rel)

    src4d = edge_index[0].reshape(NS, NBLK, BC, C)
    dst4d = edge_index[1].reshape(NS, NBLK, BC, C)
    et4d = edge_type.reshape(NS, NBLK, BC, C)
    ets4d = edge_time.reshape(NS, NBLK, BC, C)

    x2 = jnp.concatenate([x[:, :DH], x[:, DH:]], axis=0)
    rel2 = jnp.concatenate([rel_repr[:, :DH], rel_repr[:, DH:]], axis=0)
    tim2 = jnp.concatenate([time_emds[:, :DH], time_emds[:, DH:]], axis=0)

    acc, den = _sc_edge(hatt, tatt, ratt, tsatt,
                        src4d, dst4d, et4d, ets4d,
                        x2, rel2, tim2)

    x_out = _finish(acc, den, x, trans_w, loop_w)
    return (x_out, rel_out)
